# baseline (speedup 1.0000x reference)
import jax
import jax.numpy as jnp
from jax import lax
from jax.experimental import pallas as pl
from jax.experimental.pallas import tpu as pltpu
from jax.experimental.pallas import tpu_sc as plsc

R, C = 128, 100000
NW = 32
RPT = R // NW
L = 16
CH = 20000
CPR = C // CH
UNROLL = 5
STEPS = CH // (L * UNROLL)


def _shuf(x, perm):
    return lax.gather(
        x,
        perm[:, None],
        dimension_numbers=lax.GatherDimensionNumbers(
            offset_dims=(), collapsed_slice_dims=(0,), start_index_map=(0,)
        ),
        slice_sizes=(1,),
        mode=lax.GatherScatterMode.PROMISE_IN_BOUNDS,
    )


def _sc_argmax(x):
    mesh = plsc.VectorSubcoreMesh(core_axis_name="c", subcore_axis_name="s")

    @pl.kernel(
        out_type=jax.ShapeDtypeStruct((NW, L), jnp.int32),
        mesh=mesh,
        scratch_types=[
            pltpu.VMEM((CH,), jnp.float32),
            pltpu.VMEM((CH,), jnp.float32),
            pltpu.VMEM((L,), jnp.int32),
            pltpu.SemaphoreType.DMA,
            pltpu.SemaphoreType.DMA,
        ],
    )
    def body(x_hbm, out_hbm, buf0, buf1, res_v, sem0, sem1):
        wid = lax.axis_index("s") * 2 + lax.axis_index("c")
        bufs = (buf0, buf1)
        sems = (sem0, sem1)
        lanes = lax.iota(jnp.int32, L)

        def start(k):
            r, c = divmod(k, CPR)
            off = pl.multiple_of((wid * RPT + r) * C + c * CH, 8)
            return pltpu.async_copy(
                x_hbm.at[pl.ds(off, CH)], bufs[k % 2], sems[k % 2]
            )

        copies = [None, None]
        copies[0] = start(0)
        results = jnp.zeros((L,), jnp.int32)
        m = idx = None
        for k in range(RPT * CPR):
            r, c = divmod(k, CPR)
            if c == 0:
                m = jnp.full((L,), -jnp.inf, jnp.float32)
                idx = jnp.zeros((L,), jnp.int32)
            if k + 1 < RPT * CPR:
                copies[(k + 1) % 2] = start(k + 1)
            copies[k % 2].wait()
            buf = bufs[k % 2]

            def step(t, carry, buf=buf):
                m, idx, cur = carry
                for u in range(UNROLL):
                    v = buf[pl.ds(t * (L * UNROLL) + u * L, L)]
                    gt = v > m
                    m = jnp.where(gt, v, m)
                    idx = jnp.where(gt, cur, idx)
                    cur = cur + L
                return m, idx, cur

            cur0 = lanes + (c * CH)
            m, idx, _ = lax.fori_loop(0, STEPS, step, (m, idx, cur0))
            if c == CPR - 1:
                for s in (1, 2, 4, 8):
                    perm = lanes ^ s
                    mv = _shuf(m, perm)
                    iv = _shuf(idx, perm)
                    take = (mv > m) | ((mv == m) & (iv < idx))
                    m = jnp.where(take, mv, m)
                    idx = jnp.where(take, iv, idx)
                results = jnp.where(lanes == r, idx, results)
        res_v[...] = results
        pltpu.sync_copy(res_v, out_hbm.at[wid])

    return body(x)


def kernel(m_logits):
    out = _sc_argmax(m_logits.reshape(-1))
    token = out[:, :RPT].reshape(R, 1)
    return token.astype(jnp.int64)


# speedup vs baseline: 1.0394x; 1.0394x over previous
import jax
import jax.numpy as jnp
from jax import lax
from jax.experimental import pallas as pl
from jax.experimental.pallas import tpu as pltpu
from jax.experimental.pallas import tpu_sc as plsc

R, C = 128, 100000
NW = 32
RPT = R // NW
L = 16
CH = 20000
CPR = C // CH
U = 5
STEPS = CH // (L * U)
SPR = C // (L * U)


def _shuf(x, perm):
    return lax.gather(
        x,
        perm[:, None],
        dimension_numbers=lax.GatherDimensionNumbers(
            offset_dims=(), collapsed_slice_dims=(0,), start_index_map=(0,)
        ),
        slice_sizes=(1,),
        mode=lax.GatherScatterMode.PROMISE_IN_BOUNDS,
    )


def _merge(ma, ca, mb, cb):
    take = (mb > ma) | ((mb == ma) & (cb < ca))
    return jnp.where(take, mb, ma), jnp.where(take, cb, ca)


def _sc_argmax(x):
    mesh = plsc.VectorSubcoreMesh(core_axis_name="c", subcore_axis_name="s")

    @pl.kernel(
        out_type=jax.ShapeDtypeStruct((NW, L), jnp.int32),
        mesh=mesh,
        scratch_types=[
            pltpu.VMEM((CH,), jnp.float32),
            pltpu.VMEM((CH,), jnp.float32),
            pltpu.VMEM((L,), jnp.int32),
            pltpu.SemaphoreType.DMA,
            pltpu.SemaphoreType.DMA,
        ],
    )
    def body(x_hbm, out_hbm, buf0, buf1, res_v, sem0, sem1):
        wid = lax.axis_index("s") * 2 + lax.axis_index("c")
        bufs = (buf0, buf1)
        sems = (sem0, sem1)
        lanes = lax.iota(jnp.int32, L)

        def start(k):
            r, c = divmod(k, CPR)
            off = pl.multiple_of((wid * RPT + r) * C + c * CH, 8)
            return pltpu.async_copy(
                x_hbm.at[pl.ds(off, CH)], bufs[k % 2], sems[k % 2]
            )

        copies = [None, None]
        copies[0] = start(0)
        results = jnp.zeros((L,), jnp.int32)
        ms = ts = None
        for k in range(RPT * CPR):
            r, c = divmod(k, CPR)
            if c == 0:
                ms = [jnp.full((L,), -jnp.inf, jnp.float32) for _ in range(U)]
                ts = [jnp.zeros((L,), jnp.int32) for _ in range(U)]
            if k + 1 < RPT * CPR:
                copies[(k + 1) % 2] = start(k + 1)
            copies[k % 2].wait()
            buf = bufs[k % 2]

            def step(t, carry, buf=buf):
                st = list(carry)
                t_vec = st[2 * U]
                base = t * (L * U)
                for u in range(U):
                    v = buf[pl.ds(base + u * L, L)]
                    gt = v > st[u]
                    st[u] = jnp.where(gt, v, st[u])
                    st[U + u] = jnp.where(gt, t_vec, st[U + u])
                st[2 * U] = t_vec + 1
                return tuple(st)

            t0 = jnp.full((L,), c * STEPS, jnp.int32)
            out_c = lax.fori_loop(
                0, STEPS, step, tuple(ms) + tuple(ts) + (t0,)
            )
            ms, ts = list(out_c[:U]), list(out_c[U : 2 * U])
            if c == CPR - 1:
                m, col = ms[0], ts[0] * (L * U) + lanes
                for u in range(1, U):
                    m, col = _merge(
                        m, col, ms[u], ts[u] * (L * U) + (u * L) + lanes
                    )
                for s in (1, 2, 4, 8):
                    perm = lanes ^ s
                    m, col = _merge(m, col, _shuf(m, perm), _shuf(col, perm))
                results = jnp.where(lanes == r, col, results)
        res_v[...] = results
        pltpu.sync_copy(res_v, out_hbm.at[wid])

    return body(x)


def kernel(m_logits):
    out = _sc_argmax(m_logits.reshape(-1))
    token = out[:, :RPT].reshape(R, 1)
    return token.astype(jnp.int64)


# speedup vs baseline: 1.6966x; 1.6322x over previous
import jax
import jax.numpy as jnp
from jax import lax
from jax.experimental import pallas as pl
from jax.experimental.pallas import tpu as pltpu
from jax.experimental.pallas import tpu_sc as plsc

R, C = 128, 100000
L = 16
U = 4
W0 = 4992
W9 = 5120
NCH = 10
OFF9_H0 = 44800
OFF9_H1 = 94848
HSTRIDE = 49920
CTAIL = 99968
NEG_INF = float("-inf")


def _shuf(x, perm):
    return lax.gather(
        x,
        perm[:, None],
        dimension_numbers=lax.GatherDimensionNumbers(
            offset_dims=(), collapsed_slice_dims=(0,), start_index_map=(0,)
        ),
        slice_sizes=(1,),
        mode=lax.GatherScatterMode.PROMISE_IN_BOUNDS,
    )


def _merge(ma, ca, mb, cb):
    take = (mb > ma) | ((mb == ma) & (cb < ca))
    return jnp.where(take, mb, ma), jnp.where(take, cb, ca)


def _sc_argmax(x):
    mesh = plsc.VectorSubcoreMesh(core_axis_name="c", subcore_axis_name="s")

    @pl.kernel(
        out_type=(jax.ShapeDtypeStruct((R,), jnp.int32),
                  jax.ShapeDtypeStruct((512,), jnp.float32),
                  jax.ShapeDtypeStruct((512,), jnp.int32)),
        mesh=mesh,
        scratch_types=[
            pltpu.VMEM((8, W0), jnp.float32),
            pltpu.VMEM((8, W0), jnp.float32),
            pltpu.VMEM((8, W9), jnp.float32),
            pltpu.VMEM((8, 32), jnp.float32),
            pltpu.VMEM((8, L), jnp.float32),
            pltpu.VMEM((8, L), jnp.int32),
            pltpu.VMEM((L,), jnp.float32),
            pltpu.VMEM((L,), jnp.int32),
            pltpu.VMEM((L,), jnp.float32),
            pltpu.VMEM((L,), jnp.int32),
            pltpu.VMEM((L,), jnp.int32),
            pltpu.SemaphoreType.DMA,
            pltpu.SemaphoreType.DMA,
            pltpu.SemaphoreType.DMA,
        ],
    )
    def body(x_hbm, tail_hbm, out_hbm, xchm_hbm, xchc_hbm,
             bufA, bufB, bufC, tbuf, mPv, cPv,
             sm_v, sc_v, pm_v, pc_v, res_v,
             semA, semB, semC):
        s = lax.axis_index("s")
        c = lax.axis_index("c")
        h = s // 8
        g = c * 8 + (s % 8)
        row0 = pl.multiple_of(g * 8, 8)
        lanes = lax.iota(jnp.int32, L)
        bufs = (bufA, bufB)
        sems = (semA, semB)

        def chunk_off(j):
            if j < NCH - 1:
                return pl.multiple_of(h * HSTRIDE + j * W0, 128)
            return pl.multiple_of(OFF9_H0 + h * (OFF9_H1 - OFF9_H0), 128)

        def start(j):
            buf = bufs[j % 2] if j < NCH - 1 else bufC
            sem = sems[j % 2] if j < NCH - 1 else semC
            wj = W0 if j < NCH - 1 else W9
            return pltpu.async_copy(
                x_hbm.at[pl.ds(row0, 8), pl.ds(chunk_off(j), wj)], buf, sem
            )

        def init_body(r, _):
            mPv[r, :] = jnp.full((L,), NEG_INF, jnp.float32)
            cPv[r, :] = jnp.zeros((L,), jnp.int32)
            return 0

        lax.fori_loop(0, 8, init_body, 0)

        copies = [start(0), start(1), None]
        for j in range(NCH):
            buf = bufs[j % 2] if j < NCH - 1 else bufC
            copies[j % 3].wait()
            coff = chunk_off(j)
            nsteps = 78 if j < NCH - 1 else 80

            def row_body(r, _, buf=buf, coff=coff, nsteps=nsteps, j=j):
                def step(t, carry):
                    st = list(carry)
                    t_vec = st[2 * U]
                    base = t * (L * U)
                    for u in range(U):
                        v = buf[r, pl.ds(base + u * L, L)]
                        gt = v > st[u]
                        st[u] = jnp.where(gt, v, st[u])
                        st[U + u] = jnp.where(gt, t_vec, st[U + u])
                    st[2 * U] = t_vec + 1
                    return tuple(st)

                init = (
                    tuple(jnp.full((L,), NEG_INF, jnp.float32)
                          for _ in range(U))
                    + tuple(jnp.zeros((L,), jnp.int32) for _ in range(U))
                    + (jnp.zeros((L,), jnp.int32),)
                )
                out_c = lax.fori_loop(0, nsteps, step, init)
                m = out_c[0]
                col = out_c[U] * (L * U) + (lanes + coff)
                for u in range(1, U):
                    m, col = _merge(
                        m, col,
                        out_c[u],
                        out_c[U + u] * (L * U) + (lanes + (u * L + coff)),
                    )
                m, col = _merge(mPv[r, :], cPv[r, :], m, col)
                mPv[r, :] = m
                cPv[r, :] = col
                return 0

            lax.fori_loop(0, 8, row_body, 0)
            if j + 2 < NCH:
                copies[(j + 2) % 3] = start(j + 2)

        @pl.when(h == 0)
        def _():
            pltpu.sync_copy(tail_hbm.at[pl.ds(row0, 8)], tbuf)

            def tail_body(r, _):
                m = mPv[r, :]
                col = cPv[r, :]
                for e in range(2):
                    v = tbuf[r, pl.ds(e * L, L)]
                    m, col = _merge(m, col, v, lanes + (CTAIL + e * L))
                mPv[r, :] = m
                cPv[r, :] = col
                return 0

            lax.fori_loop(0, 8, tail_body, 0)

        def fin_body(r, carry):
            mAll, cAll = carry
            m = mPv[r, :]
            col = cPv[r, :]
            for sh in (1, 2, 4, 8):
                perm = lanes ^ sh
                m, col = _merge(m, col, _shuf(m, perm), _shuf(col, perm))
            sel = lanes == r
            return jnp.where(sel, m, mAll), jnp.where(sel, col, cAll)

        mAll, cAll = lax.fori_loop(
            0, 8, fin_body,
            (jnp.full((L,), NEG_INF, jnp.float32), jnp.zeros((L,), jnp.int32)),
        )

        sm_v[...] = mAll
        sc_v[...] = cAll
        wid = c * 16 + s
        doff = pl.multiple_of(wid * L, 8)
        pltpu.sync_copy(sm_v, xchm_hbm.at[pl.ds(doff, L)])
        pltpu.sync_copy(sc_v, xchc_hbm.at[pl.ds(doff, L)])
        plsc.subcore_barrier()

        @pl.when(h == 0)
        def _():
            poff = pl.multiple_of((c * 16 + s + 8) * L, 8)
            pltpu.async_copy(xchm_hbm.at[pl.ds(poff, L)], pm_v, semA).wait()
            pltpu.async_copy(xchc_hbm.at[pl.ds(poff, L)], pc_v, semB).wait()
            fm, fc = _merge(mAll, cAll, pm_v[...], pc_v[...])
            res_v[...] = fc
            pltpu.sync_copy(
                res_v.at[pl.ds(0, 8)],
                out_hbm.at[pl.ds(pl.multiple_of(g * 8, 8), 8)],
            )

    return body(x, x[:, CTAIL:])[0]


def kernel(m_logits):
    out = _sc_argmax(m_logits)
    return out.reshape(R, 1).astype(jnp.int64)


# speedup vs baseline: 2.5331x; 1.4931x over previous
import jax
import jax.numpy as jnp
from jax import lax
from jax.experimental import pallas as pl
from jax.experimental.pallas import tpu as pltpu
from jax.experimental.pallas import tpu_sc as plsc

R, C = 128, 100000
L = 16
G = 4
VW = 400
NCH = 16
WIN = VW * NCH
SHARD = C // 16
NEG_INF = float("-inf")


def _merge(ma, ia, mb, ib):
    take = (mb > ma) | ((mb == ma) & (ib < ia))
    return jnp.where(take, mb, ma), jnp.where(take, ib, ia)


def _sc_argmax(xt):
    mesh = plsc.VectorSubcoreMesh(core_axis_name="c", subcore_axis_name="s")

    @pl.kernel(
        out_type=(jax.ShapeDtypeStruct((R,), jnp.int32),
                  jax.ShapeDtypeStruct((2048,), jnp.float32),
                  jax.ShapeDtypeStruct((2048,), jnp.int32)),
        mesh=mesh,
        scratch_types=[
            pltpu.VMEM((VW, 128), jnp.float32),
            pltpu.VMEM((VW, 128), jnp.float32),
            pltpu.VMEM((G, L), jnp.float32),
            pltpu.VMEM((G, L), jnp.int32),
            pltpu.VMEM((L * L,), jnp.float32),
            pltpu.VMEM((L * L,), jnp.int32),
            pltpu.VMEM((L,), jnp.int32),
            pltpu.SemaphoreType.DMA,
            pltpu.SemaphoreType.DMA,
            pltpu.SemaphoreType.DMA,
        ],
    )
    def body(x_hbm, out_hbm, xchm_hbm, xchc_hbm,
             bufA, bufB, mv, iv, gm, gi, res_v, semA, semB, semC):
        s = lax.axis_index("s")
        c = lax.axis_index("c")
        bufs = (bufA, bufB)
        sems = (semA, semB)
        off0 = (s * SHARD) // 8 * 8
        off0 = pl.multiple_of(jnp.minimum(off0, C - WIN), 8)

        def start(j):
            return pltpu.async_copy(
                x_hbm.at[pl.ds(pl.multiple_of(off0 + j * VW, 8), VW)],
                bufs[j % 2], sems[j % 2])

        copies = [start(0), start(1)]
        ms = [jnp.full((L,), NEG_INF, jnp.float32) for _ in range(G)]
        is_ = [jnp.zeros((L,), jnp.int32) for _ in range(G)]
        for j in range(NCH):
            buf = bufs[j % 2]
            copies[j % 2].wait()
            t0 = jnp.zeros((L,), jnp.int32) + (off0 + j * VW)

            def step(v, carry, buf=buf):
                st = list(carry)
                t_vec = st[2 * G]
                for g in range(G):
                    val = buf[v, pl.ds((c * G + g) * L, L)]
                    gt = val > st[g]
                    st[g] = jnp.where(gt, val, st[g])
                    st[G + g] = jnp.where(gt, t_vec, st[G + g])
                st[2 * G] = t_vec + 1
                return tuple(st)

            out_c = lax.fori_loop(0, VW, step, tuple(ms) + tuple(is_) + (t0,))
            ms, is_ = list(out_c[:G]), list(out_c[G:2 * G])
            if j + 2 < NCH:
                copies[j % 2] = start(j + 2)

        for g in range(G):
            mv[g, :] = ms[g]
            iv[g, :] = is_[g]
        for g in range(G):
            doff = pl.multiple_of(((c * G + g) * L + s) * L, 8)
            pltpu.sync_copy(mv.at[g], xchm_hbm.at[pl.ds(doff, L)])
            pltpu.sync_copy(iv.at[g], xchc_hbm.at[pl.ds(doff, L)])
        plsc.subcore_barrier()

        @pl.when(s < G)
        def _():
            poff = pl.multiple_of((c * G + s) * L * L, 8)
            pltpu.async_copy(xchm_hbm.at[pl.ds(poff, L * L)], gm, semC).wait()
            pltpu.async_copy(xchc_hbm.at[pl.ds(poff, L * L)], gi, semC).wait()
            m, idx = gm[pl.ds(0, L)], gi[pl.ds(0, L)]
            for t in range(1, L):
                m, idx = _merge(m, idx,
                                gm[pl.ds(t * L, L)], gi[pl.ds(t * L, L)])
            res_v[...] = idx
            pltpu.sync_copy(
                res_v,
                out_hbm.at[pl.ds(pl.multiple_of((c * G + s) * L, 8), L)])

    return body(xt)


def kernel(m_logits):
    out = _sc_argmax(m_logits.T)[0]
    return out.reshape(R, 1).astype(jnp.int64)
